# baseline (speedup 1.0000x reference)
import jax
import jax.numpy as jnp
from jax.experimental import pallas as pl
from jax.experimental.pallas import tpu as pltpu

NUM_LAYERS = 4
D = 1024
OUT = 64
TM = 512
NSUB = 4
SM = TM // NSUB


def _moe_kernel(x_ref, wr_ref, wl_ref, bl_ref, we_ref, be_ref, out_ref,
                w0_ref):
    @pl.when(pl.program_id(0) == 0)
    def _build_w0():
        w0_ref[...] = wl_ref[0, :D, :] + wl_ref[0, D:, :]

    for s in range(NSUB):
        r = slice(s * SM, (s + 1) * SM)
        x = x_ref[r, :]
        logits = jnp.dot(x, wr_ref[...], preferred_element_type=jnp.float32)
        g = jax.nn.softmax(logits, axis=-1)
        if s == 0:
            xs, gs = [x], [g]
        else:
            xs.append(x)
            gs.append(g)

    curs = list(xs)
    accs = [jnp.zeros((SM, OUT), dtype=jnp.float32) for _ in range(NSUB)]
    keeps = [jnp.ones((SM, 1), dtype=jnp.bool_) for _ in range(NSUB)]
    for d in range(NUM_LAYERS):
        hs = []
        for s in range(NSUB):
            if d == 0:
                h = jnp.dot(xs[s], w0_ref[...],
                            preferred_element_type=jnp.float32)
            else:
                h = jnp.dot(curs[s], wl_ref[d, :D, :],
                            preferred_element_type=jnp.float32)
                h = h + jnp.dot(xs[s], wl_ref[d, D:, :],
                                preferred_element_type=jnp.float32)
            hs.append(h)
        for s in range(NSUB):
            h = jnp.maximum(hs[s] + bl_ref[d:d + 1, :], 0.0)
            curs[s] = curs[s] + h
        ps = []
        for s in range(NSUB):
            p = jnp.dot(curs[s], we_ref[d], preferred_element_type=jnp.float32)
            ps.append(p + be_ref[d:d + 1, :])
        for s in range(NSUB):
            g, p = gs[s], ps[s]
            if d < NUM_LAYERS - 1:
                raw0 = g[:, d:d + 1]
                raw1 = g[:, d + 1:d + 2]
                for j in range(d + 2, NUM_LAYERS):
                    raw1 = raw1 + g[:, j:j + 1]
                denom = jnp.abs(raw0) + jnp.abs(raw1)
                mask_exit = (raw0 / denom) > 0.0
                mask_enter = (raw1 / denom) > 0.0
                accs[s] = accs[s] + jnp.where(
                    jnp.logical_and(keeps[s], mask_exit), p, 0.0)
                keeps[s] = jnp.logical_and(keeps[s], mask_enter)
            else:
                accs[s] = accs[s] + jnp.where(keeps[s], p, 0.0)

    for s in range(NSUB):
        out_ref[slice(s * SM, (s + 1) * SM), :] = accs[s]


def kernel(inputs, Wr, Wl, bl, We, be):
    n_tokens = inputs.shape[0]
    return pl.pallas_call(
        _moe_kernel,
        grid=(n_tokens // TM,),
        in_specs=[
            pl.BlockSpec((TM, D), lambda i: (i, 0)),
            pl.BlockSpec((D, NUM_LAYERS), lambda i: (0, 0)),
            pl.BlockSpec((NUM_LAYERS, 2 * D, D), lambda i: (0, 0, 0)),
            pl.BlockSpec((NUM_LAYERS, D), lambda i: (0, 0)),
            pl.BlockSpec((NUM_LAYERS, D, OUT), lambda i: (0, 0, 0)),
            pl.BlockSpec((NUM_LAYERS, OUT), lambda i: (0, 0)),
        ],
        out_specs=pl.BlockSpec((TM, OUT), lambda i: (i, 0)),
        out_shape=jax.ShapeDtypeStruct((n_tokens, OUT), jnp.float32),
        scratch_shapes=[pltpu.VMEM((D, D), jnp.float32)],
    )(inputs, Wr, Wl, bl, We, be)


# speedup vs baseline: 1.0495x; 1.0495x over previous
import jax
import jax.numpy as jnp
from jax.experimental import pallas as pl
from jax.experimental.pallas import tpu as pltpu

NUM_LAYERS = 4
D = 1024
OUT = 64
TM = 1024
NSUB = 4
SM = TM // NSUB


def _moe_kernel(x_ref, wr_ref, wl_ref, bl_ref, we_ref, be_ref, out_ref,
                w0_ref):
    @pl.when(pl.program_id(0) == 0)
    def _build_w0():
        w0_ref[...] = (wl_ref[0, :D, :] + wl_ref[0, D:, :]).astype(
            jnp.bfloat16)

    for s in range(NSUB):
        r = slice(s * SM, (s + 1) * SM)
        x = x_ref[r, :]
        logits = jnp.dot(x, wr_ref[...], preferred_element_type=jnp.float32)
        g = jax.nn.softmax(logits, axis=-1)
        if s == 0:
            xs, gs = [x], [g]
        else:
            xs.append(x)
            gs.append(g)

    curs = list(xs)
    accs = [jnp.zeros((SM, OUT), dtype=jnp.float32) for _ in range(NSUB)]
    keeps = [jnp.ones((SM, 1), dtype=jnp.bool_) for _ in range(NSUB)]
    for d in range(NUM_LAYERS):
        hs = []
        for s in range(NSUB):
            if d == 0:
                h = jnp.dot(xs[s].astype(jnp.bfloat16), w0_ref[...],
                            preferred_element_type=jnp.float32)
            else:
                h = jnp.dot(curs[s], wl_ref[d, :D, :],
                            preferred_element_type=jnp.float32)
                h = h + jnp.dot(xs[s], wl_ref[d, D:, :],
                                preferred_element_type=jnp.float32)
            hs.append(h)
        for s in range(NSUB):
            h = jnp.maximum(hs[s] + bl_ref[d:d + 1, :], 0.0)
            curs[s] = curs[s] + h
        ps = []
        for s in range(NSUB):
            p = jnp.dot(curs[s], we_ref[d], preferred_element_type=jnp.float32)
            ps.append(p + be_ref[d:d + 1, :])
        for s in range(NSUB):
            g, p = gs[s], ps[s]
            if d < NUM_LAYERS - 1:
                raw0 = g[:, d:d + 1]
                raw1 = g[:, d + 1:d + 2]
                for j in range(d + 2, NUM_LAYERS):
                    raw1 = raw1 + g[:, j:j + 1]
                denom = jnp.abs(raw0) + jnp.abs(raw1)
                mask_exit = (raw0 / denom) > 0.0
                mask_enter = (raw1 / denom) > 0.0
                accs[s] = accs[s] + jnp.where(
                    jnp.logical_and(keeps[s], mask_exit), p, 0.0)
                keeps[s] = jnp.logical_and(keeps[s], mask_enter)
            else:
                accs[s] = accs[s] + jnp.where(keeps[s], p, 0.0)

    for s in range(NSUB):
        out_ref[slice(s * SM, (s + 1) * SM), :] = accs[s]


def kernel(inputs, Wr, Wl, bl, We, be):
    n_tokens = inputs.shape[0]
    return pl.pallas_call(
        _moe_kernel,
        grid=(n_tokens // TM,),
        in_specs=[
            pl.BlockSpec((TM, D), lambda i: (i, 0)),
            pl.BlockSpec((D, NUM_LAYERS), lambda i: (0, 0)),
            pl.BlockSpec((NUM_LAYERS, 2 * D, D), lambda i: (0, 0, 0)),
            pl.BlockSpec((NUM_LAYERS, D), lambda i: (0, 0)),
            pl.BlockSpec((NUM_LAYERS, D, OUT), lambda i: (0, 0, 0)),
            pl.BlockSpec((NUM_LAYERS, OUT), lambda i: (0, 0)),
        ],
        out_specs=pl.BlockSpec((TM, OUT), lambda i: (i, 0)),
        out_shape=jax.ShapeDtypeStruct((n_tokens, OUT), jnp.float32),
        scratch_shapes=[pltpu.VMEM((D, D), jnp.bfloat16)],
    )(inputs, Wr, Wl, bl, We, be)


# speedup vs baseline: 1.0549x; 1.0052x over previous
import jax
import jax.numpy as jnp
from jax.experimental import pallas as pl
from jax.experimental.pallas import tpu as pltpu

NUM_LAYERS = 4
D = 1024
OUT = 64
TM = 1024
NSUB = 2
SM = TM // NSUB


def _moe_kernel(x_ref, wr_ref, wl_ref, bl_ref, we_ref, be_ref, out_ref,
                w0_ref):
    @pl.when(pl.program_id(0) == 0)
    def _build_w0():
        w0_ref[...] = (wl_ref[0, :D, :] + wl_ref[0, D:, :]).astype(
            jnp.bfloat16)

    for s in range(NSUB):
        r = slice(s * SM, (s + 1) * SM)
        x = x_ref[r, :]
        logits = jnp.dot(x, wr_ref[...], preferred_element_type=jnp.float32)
        g = jax.nn.softmax(logits, axis=-1)
        if s == 0:
            xs, gs = [x], [g]
        else:
            xs.append(x)
            gs.append(g)

    curs = list(xs)
    accs = [jnp.zeros((SM, OUT), dtype=jnp.float32) for _ in range(NSUB)]
    keeps = [jnp.ones((SM, 1), dtype=jnp.bool_) for _ in range(NSUB)]
    for d in range(NUM_LAYERS):
        hs = []
        for s in range(NSUB):
            if d == 0:
                h = jnp.dot(xs[s].astype(jnp.bfloat16), w0_ref[...],
                            preferred_element_type=jnp.float32)
            else:
                h = jnp.dot(curs[s], wl_ref[d, :D, :],
                            preferred_element_type=jnp.float32)
                h = h + jnp.dot(xs[s], wl_ref[d, D:, :],
                                preferred_element_type=jnp.float32)
            hs.append(h)
        for s in range(NSUB):
            h = jnp.maximum(hs[s] + bl_ref[d:d + 1, :], 0.0)
            curs[s] = curs[s] + h
        ps = []
        for s in range(NSUB):
            p = jnp.dot(curs[s], we_ref[d], preferred_element_type=jnp.float32)
            ps.append(p + be_ref[d:d + 1, :])
        for s in range(NSUB):
            g, p = gs[s], ps[s]
            if d < NUM_LAYERS - 1:
                raw0 = g[:, d:d + 1]
                raw1 = g[:, d + 1:d + 2]
                for j in range(d + 2, NUM_LAYERS):
                    raw1 = raw1 + g[:, j:j + 1]
                denom = jnp.abs(raw0) + jnp.abs(raw1)
                mask_exit = (raw0 / denom) > 0.0
                mask_enter = (raw1 / denom) > 0.0
                accs[s] = accs[s] + jnp.where(
                    jnp.logical_and(keeps[s], mask_exit), p, 0.0)
                keeps[s] = jnp.logical_and(keeps[s], mask_enter)
            else:
                accs[s] = accs[s] + jnp.where(keeps[s], p, 0.0)

    for s in range(NSUB):
        out_ref[slice(s * SM, (s + 1) * SM), :] = accs[s]


def kernel(inputs, Wr, Wl, bl, We, be):
    n_tokens = inputs.shape[0]
    return pl.pallas_call(
        _moe_kernel,
        grid=(n_tokens // TM,),
        in_specs=[
            pl.BlockSpec((TM, D), lambda i: (i, 0)),
            pl.BlockSpec((D, NUM_LAYERS), lambda i: (0, 0)),
            pl.BlockSpec((NUM_LAYERS, 2 * D, D), lambda i: (0, 0, 0)),
            pl.BlockSpec((NUM_LAYERS, D), lambda i: (0, 0)),
            pl.BlockSpec((NUM_LAYERS, D, OUT), lambda i: (0, 0, 0)),
            pl.BlockSpec((NUM_LAYERS, OUT), lambda i: (0, 0)),
        ],
        out_specs=pl.BlockSpec((TM, OUT), lambda i: (i, 0)),
        out_shape=jax.ShapeDtypeStruct((n_tokens, OUT), jnp.float32),
        scratch_shapes=[pltpu.VMEM((D, D), jnp.bfloat16)],
    )(inputs, Wr, Wl, bl, We, be)
